# baseline (speedup 1.0000x reference)
import functools

import jax
import jax.numpy as jnp
from jax import lax
from jax.experimental import pallas as pl
from jax.experimental.pallas import tpu as pltpu
from jax.experimental.pallas import tpu_sc as plsc

_NC = 2
_NS = 16
_NW = _NC * _NS
_C = 80


def _premul(x, ws_t, wr_t):
    n, d = x.shape

    def body(x_ref, ws_ref, wr_ref, a_ref, b_ref):
        xv = x_ref[...]
        a_ref[...] = jnp.dot(xv, ws_ref[...], preferred_element_type=jnp.float32)
        b_ref[...] = jnp.dot(xv, wr_ref[...], preferred_element_type=jnp.float32)

    return pl.pallas_call(
        body,
        out_shape=[jax.ShapeDtypeStruct((n, d), jnp.float32),
                   jax.ShapeDtypeStruct((n, d), jnp.float32)],
    )(x, ws_t, wr_t)


_CC = 128


def _sc_gather(xw_s, xw_r, send, recv):
    e = send.shape[0]
    d = xw_s.shape[1]
    ew = e // _NW
    nfull = ew // _CC
    tailc = ew - nfull * _CC
    npairs = nfull // 2
    assert e % _NW == 0 and nfull % 2 == 0 and tailc % 8 == 0
    mesh = plsc.VectorSubcoreMesh(core_axis_name="c", subcore_axis_name="s")

    @functools.partial(
        pl.kernel,
        out_type=[jax.ShapeDtypeStruct((e, d), jnp.float32),
                  jax.ShapeDtypeStruct((e, d), jnp.float32)],
        mesh=mesh,
        scratch_types=[
            pltpu.VMEM((2, _CC), jnp.int32),
            pltpu.VMEM((2, _CC), jnp.int32),
            pltpu.VMEM((2, _CC, d), jnp.float32),
            pltpu.VMEM((2, _CC, d), jnp.float32),
            pltpu.SemaphoreType.DMA,
            pltpu.SemaphoreType.DMA,
            pltpu.SemaphoreType.DMA,
            pltpu.SemaphoreType.DMA,
        ],
    )
    def k(xs_hbm, xr_hbm, send_hbm, recv_hbm, a_hbm, b_hbm,
          idx_s, idx_r, buf_a, buf_b, sem_i, sem_g, sem_w0, sem_w1):
        wid = lax.axis_index("s") * _NC + lax.axis_index("c")
        base0 = pl.multiple_of(wid * ew, 8)
        semw = (sem_w0, sem_w1)

        def jbase(j):
            return pl.multiple_of(base0 + j * _CC, 8)

        def idx_load(j, sl):
            pltpu.async_copy(send_hbm.at[pl.ds(jbase(j), _CC)], idx_s.at[sl], sem_i)
            pltpu.async_copy(recv_hbm.at[pl.ds(jbase(j), _CC)], idx_r.at[sl], sem_i)

        def idx_wait(sl):
            pltpu.make_async_copy(send_hbm.at[pl.ds(base0, _CC)], idx_s.at[sl], sem_i).wait()
            pltpu.make_async_copy(recv_hbm.at[pl.ds(base0, _CC)], idx_r.at[sl], sem_i).wait()

        def gather_issue(sl):
            pltpu.async_copy(xs_hbm.at[idx_s.at[sl]], buf_a.at[sl], sem_g)
            pltpu.async_copy(xr_hbm.at[idx_r.at[sl]], buf_b.at[sl], sem_g)

        def gather_wait(sl):
            pltpu.make_async_copy(xs_hbm.at[idx_s.at[sl]], buf_a.at[sl], sem_g).wait()
            pltpu.make_async_copy(xr_hbm.at[idx_r.at[sl]], buf_b.at[sl], sem_g).wait()

        def wb_issue(j, sl):
            pltpu.async_copy(buf_a.at[sl], a_hbm.at[pl.ds(jbase(j), _CC)], semw[sl])
            pltpu.async_copy(buf_b.at[sl], b_hbm.at[pl.ds(jbase(j), _CC)], semw[sl])

        def wb_wait(sl):
            pltpu.make_async_copy(buf_a.at[sl], a_hbm.at[pl.ds(base0, _CC)], semw[sl]).wait()
            pltpu.make_async_copy(buf_b.at[sl], b_hbm.at[pl.ds(base0, _CC)], semw[sl]).wait()

        idx_load(0, 0)
        idx_wait(0)
        gather_issue(0)
        idx_load(1, 1)

        def pair(g, carry):
            j0 = 2 * g

            @pl.when(g > 0)
            def _():
                idx_load(j0 + 1, 1)
            gather_wait(0)
            wb_issue(j0, 0)
            idx_wait(1)

            @pl.when(g > 0)
            def _():
                wb_wait(1)
            gather_issue(1)

            @pl.when(g < npairs - 1)
            def _():
                idx_load(j0 + 2, 0)
            gather_wait(1)
            wb_issue(j0 + 1, 1)

            @pl.when(g < npairs - 1)
            def _():
                idx_wait(0)
                wb_wait(0)
                gather_issue(0)
            return carry

        lax.fori_loop(0, npairs, pair, 0)
        wb_wait(1)
        wb_wait(0)
        if tailc:
            tb = pl.multiple_of(base0 + nfull * _CC, 8)
            pltpu.sync_copy(send_hbm.at[pl.ds(tb, tailc)], idx_s.at[0, pl.ds(0, tailc)])
            pltpu.sync_copy(recv_hbm.at[pl.ds(tb, tailc)], idx_r.at[0, pl.ds(0, tailc)])
            cpa = pltpu.async_copy(xs_hbm.at[idx_s.at[0, pl.ds(0, tailc)]],
                                   buf_a.at[0, pl.ds(0, tailc)], sem_g)
            cpb = pltpu.async_copy(xr_hbm.at[idx_r.at[0, pl.ds(0, tailc)]],
                                   buf_b.at[0, pl.ds(0, tailc)], sem_g)
            cpa.wait()
            cpb.wait()
            pltpu.sync_copy(buf_a.at[0, pl.ds(0, tailc)], a_hbm.at[pl.ds(tb, tailc)])
            pltpu.sync_copy(buf_b.at[0, pl.ds(0, tailc)], b_hbm.at[pl.ds(tb, tailc)])

    return k(xw_s, xw_r, send, recv)


def _sc_counts(recv, zeros_nd, ones_cd):
    e = recv.shape[0]
    n, d = zeros_nd.shape
    ew = e // _NW
    nfull = ew // _CC
    tailc = ew - nfull * _CC
    npairs = nfull // 2
    rt = (n // _NS) & ~7
    tail = n - rt * _NS
    assert nfull % 2 == 0 and tailc % 8 == 0 and rt % 8 == 0 and tail % 8 == 0
    mesh = plsc.VectorSubcoreMesh(core_axis_name="c", subcore_axis_name="s")

    @functools.partial(
        pl.kernel,
        out_type=jax.ShapeDtypeStruct((_NC * n, d), jnp.float32),
        mesh=mesh,
        scratch_types=[
            pltpu.VMEM((2, _CC), jnp.int32),
            pltpu.VMEM((_CC, d), jnp.float32),
            pltpu.VMEM_SHARED((n + 16, d), jnp.float32),
            pltpu.SemaphoreType.DMA,
            pltpu.SemaphoreType.DMA,
        ],
    )
    def k(recv_hbm, zs_hbm, ones_hbm, pcnt_hbm, idx2, ones_v, cacc, sem0, sem1):
        c = lax.axis_index("c")
        s = lax.axis_index("s")
        wid = s * _NC + c
        base0 = pl.multiple_of(wid * ew, 8)
        row0 = pl.multiple_of(s * rt, 8)
        out0 = pl.multiple_of(c * n, 8)
        sems = (sem0, sem1)

        def each_acc_slab(fn):
            fn(row0, rt)
            if tail:
                @pl.when(s == _NS - 1)
                def _():
                    fn(rt * _NS, tail)

        each_acc_slab(lambda r0, sz: pltpu.sync_copy(
            zs_hbm.at[pl.ds(r0, sz)], cacc.at[pl.ds(r0, sz)]))
        pltpu.sync_copy(ones_hbm, ones_v)
        plsc.subcore_barrier()

        def idx_load(j, sl):
            pltpu.async_copy(recv_hbm.at[pl.ds(pl.multiple_of(base0 + j * _CC, 8), _CC)],
                             idx2.at[sl], sems[sl])

        def idx_wait(sl):
            pltpu.make_async_copy(recv_hbm.at[pl.ds(base0, _CC)], idx2.at[sl], sems[sl]).wait()

        def scat(sl):
            pltpu.sync_copy(ones_v, cacc.at[idx2.at[sl]], add=True)

        idx_load(0, 0)

        def pair(g, carry):
            j0 = 2 * g
            idx_load(j0 + 1, 1)
            idx_wait(0)
            scat(0)

            @pl.when(g < npairs - 1)
            def _():
                idx_load(j0 + 2, 0)
            idx_wait(1)
            scat(1)
            return carry

        lax.fori_loop(0, npairs, pair, 0)
        if tailc:
            tb = pl.multiple_of(base0 + nfull * _CC, 8)
            pltpu.sync_copy(recv_hbm.at[pl.ds(tb, tailc)], idx2.at[0, pl.ds(0, tailc)])
            for t in range(tailc // 16, _CC // 16):
                idx2[0, pl.ds(t * 16, 16)] = jnp.full((16,), n, jnp.int32)
            scat(0)
        plsc.subcore_barrier()
        each_acc_slab(lambda r0, sz: pltpu.sync_copy(
            cacc.at[pl.ds(r0, sz)], pcnt_hbm.at[pl.ds(out0 + r0, sz)]))

    return k(recv, zeros_nd, ones_cd).reshape(_NC, n, d)


def _edge_mlp(a, b, ea, w1e_t, b1, w2_t, b2):
    e, d = a.shape
    be = 1280
    assert e % be == 0

    def body(a_ref, b_ref, ea_ref, w1e_ref, b1_ref, w2_ref, b2_ref, h_ref):
        ea16 = ea_ref[...].astype(jnp.bfloat16)
        t = (a_ref[...] + b_ref[...] + b1_ref[...]
             + jnp.dot(ea16, w1e_ref[...], preferred_element_type=jnp.float32))
        t = t * jax.nn.sigmoid(t)
        t2 = (jnp.dot(t.astype(jnp.bfloat16), w2_ref[...],
                      preferred_element_type=jnp.float32) + b2_ref[...])
        h_ref[...] = t2 * jax.nn.sigmoid(t2)

    row = lambda i: (i, 0)
    rep = lambda i: (0, 0)
    return pl.pallas_call(
        body,
        grid=(e // be,),
        in_specs=[
            pl.BlockSpec((be, d), row),
            pl.BlockSpec((be, d), row),
            pl.BlockSpec((be, d), row),
            pl.BlockSpec((d, d), rep),
            pl.BlockSpec((1, d), rep),
            pl.BlockSpec((d, d), rep),
            pl.BlockSpec((1, d), rep),
        ],
        out_specs=pl.BlockSpec((be, d), row),
        out_shape=jax.ShapeDtypeStruct((e, d), jnp.float32),
        compiler_params=pltpu.CompilerParams(
            dimension_semantics=("arbitrary",)),
    )(a, b, ea, w1e_t.astype(jnp.bfloat16), b1.reshape(1, d),
      w2_t.astype(jnp.bfloat16), b2.reshape(1, d))


def _sc_scatter(h, recv, zsum):
    e, d = h.shape
    n = zsum.shape[0]
    ew = e // _NW
    rt = (n // _NS) & ~7
    tail = n - rt * _NS
    assert e % _NW == 0 and ew % _C == 0 and rt % 8 == 0 and tail % 8 == 0
    mesh = plsc.VectorSubcoreMesh(core_axis_name="c", subcore_axis_name="s")

    nfull = ew // _CC
    tailc = ew - nfull * _CC
    npairs = nfull // 2
    assert nfull % 2 == 0 and tailc % 8 == 0

    @functools.partial(
        pl.kernel,
        out_type=jax.ShapeDtypeStruct((_NC * n, d), jnp.float32),
        mesh=mesh,
        scratch_types=[
            pltpu.VMEM((2, _CC), jnp.int32),
            pltpu.VMEM((2, _CC, d), jnp.float32),
            pltpu.VMEM_SHARED((n + 16, d), jnp.float32),
            pltpu.SemaphoreType.DMA,
            pltpu.SemaphoreType.DMA,
        ],
    )
    def k(h_hbm, recv_hbm, zs_hbm, psum_hbm, idx2, buf, acc, sem0, sem1):
        c = lax.axis_index("c")
        s = lax.axis_index("s")
        wid = s * _NC + c
        base0 = pl.multiple_of(wid * ew, 8)
        row0 = pl.multiple_of(s * rt, 8)
        out0 = pl.multiple_of(c * n, 8)
        sems = (sem0, sem1)

        def each_acc_slab(fn):
            fn(row0, rt)
            if tail:
                @pl.when(s == _NS - 1)
                def _():
                    fn(rt * _NS, tail)

        each_acc_slab(lambda r0, sz: pltpu.sync_copy(
            zs_hbm.at[pl.ds(r0, sz)], acc.at[pl.ds(r0, sz)]))
        plsc.subcore_barrier()

        def load(j, sl):
            base = pl.multiple_of(base0 + j * _CC, 8)
            pltpu.async_copy(recv_hbm.at[pl.ds(base, _CC)], idx2.at[sl], sems[sl])
            pltpu.async_copy(h_hbm.at[pl.ds(base, _CC)], buf.at[sl], sems[sl])

        def load_wait(sl):
            pltpu.make_async_copy(recv_hbm.at[pl.ds(base0, _CC)], idx2.at[sl], sems[sl]).wait()
            pltpu.make_async_copy(h_hbm.at[pl.ds(base0, _CC)], buf.at[sl], sems[sl]).wait()

        def scat(sl):
            pltpu.sync_copy(buf.at[sl], acc.at[idx2.at[sl]], add=True)

        load(0, 0)

        def pair(g, carry):
            j0 = 2 * g
            load(j0 + 1, 1)
            load_wait(0)
            scat(0)

            @pl.when(g < npairs - 1)
            def _():
                load(j0 + 2, 0)
            load_wait(1)
            scat(1)
            return carry

        lax.fori_loop(0, npairs, pair, 0)
        if tailc:
            tb = pl.multiple_of(base0 + nfull * _CC, 8)
            pltpu.sync_copy(recv_hbm.at[pl.ds(tb, tailc)], idx2.at[0, pl.ds(0, tailc)])
            pltpu.sync_copy(h_hbm.at[pl.ds(tb, tailc)], buf.at[0, pl.ds(0, tailc)])
            for t in range(tailc // 16, _CC // 16):
                idx2[0, pl.ds(t * 16, 16)] = jnp.full((16,), n, jnp.int32)
            scat(0)
        plsc.subcore_barrier()
        each_acc_slab(lambda r0, sz: pltpu.sync_copy(
            acc.at[pl.ds(r0, sz)], psum_hbm.at[pl.ds(out0 + r0, sz)]))

    return k(h, recv, zsum).reshape(_NC, n, d)


def _update(x, psum, pcnt, u1_t, ub1, u2_t, ub2):
    n, d = x.shape

    def body(x_ref, ps_ref, pc_ref, u1_ref, ub1_ref, u2_ref, ub2_ref, out_ref):
        cnt = pc_ref[0, :, 0:1] + pc_ref[1, :, 0:1]
        mean = (ps_ref[0] + ps_ref[1]) / jnp.maximum(cnt, 1.0)
        x_new = x_ref[...] + mean
        t = jnp.dot(x_new, u1_ref[...], preferred_element_type=jnp.float32) + ub1_ref[...]
        t = t * jax.nn.sigmoid(t)
        u = jnp.dot(t, u2_ref[...], preferred_element_type=jnp.float32) + ub2_ref[...]
        out_ref[...] = x_new + u

    return pl.pallas_call(
        body,
        out_shape=jax.ShapeDtypeStruct((n, d), jnp.float32),
    )(x, psum, pcnt, u1_t, ub1.reshape(1, -1), u2_t, ub2.reshape(1, -1))


def kernel(x, edge_attr, edges, W1, b1, W2, b2, U1, ub1, U2, ub2):
    n, d = x.shape
    send = edges[0]
    recv = edges[1]
    w1s_t = W1[:, :d].T
    w1r_t = W1[:, d:2 * d].T
    w1e_t = W1[:, 2 * d:].T
    w2_t = W2.T
    u1_t = U1.T
    u2_t = U2.T

    xw_s, xw_r = _premul(x, w1s_t, w1r_t)
    zsum = jnp.zeros((n, d), jnp.float32)
    ones_cd = jnp.ones((_CC, d), jnp.float32)
    a, b = _sc_gather(xw_s, xw_r, send, recv)
    h = _edge_mlp(a, b, edge_attr, w1e_t, b1, w2_t, b2)
    pcnt = _sc_counts(recv, zsum, ones_cd)
    psum = _sc_scatter(h, recv, zsum)
    x_out = _update(x, psum, pcnt, u1_t, ub1, u2_t, ub2)
    return (x_out, h)


# speedup vs baseline: 1.0074x; 1.0074x over previous
import functools

import jax
import jax.numpy as jnp
from jax import lax
from jax.experimental import pallas as pl
from jax.experimental.pallas import tpu as pltpu
from jax.experimental.pallas import tpu_sc as plsc

_NC = 2
_NS = 16
_NW = _NC * _NS
_C = 80


def _premul(x, ws_t, wr_t):
    n, d = x.shape

    def body(x_ref, ws_ref, wr_ref, a_ref, b_ref):
        xv = x_ref[...]
        a_ref[...] = jnp.dot(xv, ws_ref[...], preferred_element_type=jnp.float32)
        b_ref[...] = jnp.dot(xv, wr_ref[...], preferred_element_type=jnp.float32)

    return pl.pallas_call(
        body,
        out_shape=[jax.ShapeDtypeStruct((n, d), jnp.float32),
                   jax.ShapeDtypeStruct((n, d), jnp.float32)],
    )(x, ws_t, wr_t)


_CC = 128


def _sc_gather(xw_s, xw_r, send, recv):
    e = send.shape[0]
    d = xw_s.shape[1]
    ew = e // _NW
    nfull = ew // _CC
    tailc = ew - nfull * _CC
    npairs = nfull // 2
    assert e % _NW == 0 and nfull % 2 == 0 and tailc % 8 == 0
    mesh = plsc.VectorSubcoreMesh(core_axis_name="c", subcore_axis_name="s")

    @functools.partial(
        pl.kernel,
        out_type=[jax.ShapeDtypeStruct((e, d), jnp.float32),
                  jax.ShapeDtypeStruct((e, d), jnp.float32)],
        mesh=mesh,
        scratch_types=[
            pltpu.VMEM((2, _CC), jnp.int32),
            pltpu.VMEM((2, _CC), jnp.int32),
            pltpu.VMEM((2, _CC, d), jnp.float32),
            pltpu.VMEM((2, _CC, d), jnp.float32),
            pltpu.SemaphoreType.DMA,
            pltpu.SemaphoreType.DMA,
            pltpu.SemaphoreType.DMA,
            pltpu.SemaphoreType.DMA,
            pltpu.SemaphoreType.DMA,
        ],
    )
    def k(xs_hbm, xr_hbm, send_hbm, recv_hbm, a_hbm, b_hbm,
          idx_s, idx_r, buf_a, buf_b, sem_i, sem_g0, sem_g1, sem_w0, sem_w1):
        wid = lax.axis_index("s") * _NC + lax.axis_index("c")
        base0 = pl.multiple_of(wid * ew, 8)
        semw = (sem_w0, sem_w1)
        semg = (sem_g0, sem_g1)

        def jbase(j):
            return pl.multiple_of(base0 + j * _CC, 8)

        def idx_load(j, sl):
            pltpu.async_copy(send_hbm.at[pl.ds(jbase(j), _CC)], idx_s.at[sl], sem_i)
            pltpu.async_copy(recv_hbm.at[pl.ds(jbase(j), _CC)], idx_r.at[sl], sem_i)

        def idx_wait(sl):
            pltpu.make_async_copy(send_hbm.at[pl.ds(base0, _CC)], idx_s.at[sl], sem_i).wait()
            pltpu.make_async_copy(recv_hbm.at[pl.ds(base0, _CC)], idx_r.at[sl], sem_i).wait()

        def gather_issue(sl):
            pltpu.async_copy(xs_hbm.at[idx_s.at[sl]], buf_a.at[sl], semg[sl])
            pltpu.async_copy(xr_hbm.at[idx_r.at[sl]], buf_b.at[sl], semg[sl])

        def gather_wait(sl):
            pltpu.make_async_copy(xs_hbm.at[idx_s.at[sl]], buf_a.at[sl], semg[sl]).wait()
            pltpu.make_async_copy(xr_hbm.at[idx_r.at[sl]], buf_b.at[sl], semg[sl]).wait()

        def wb_issue(j, sl):
            pltpu.async_copy(buf_a.at[sl], a_hbm.at[pl.ds(jbase(j), _CC)], semw[sl])
            pltpu.async_copy(buf_b.at[sl], b_hbm.at[pl.ds(jbase(j), _CC)], semw[sl])

        def wb_wait(sl):
            pltpu.make_async_copy(buf_a.at[sl], a_hbm.at[pl.ds(base0, _CC)], semw[sl]).wait()
            pltpu.make_async_copy(buf_b.at[sl], b_hbm.at[pl.ds(base0, _CC)], semw[sl]).wait()

        idx_load(0, 0)
        idx_wait(0)
        gather_issue(0)
        idx_load(1, 1)

        def pair(g, carry):
            j0 = 2 * g
            idx_wait(1)

            @pl.when(g > 0)
            def _():
                wb_wait(1)
            gather_issue(1)
            gather_wait(0)
            wb_issue(j0, 0)

            @pl.when(g < npairs - 1)
            def _():
                idx_load(j0 + 2, 0)
                idx_wait(0)
                wb_wait(0)
                gather_issue(0)
            gather_wait(1)
            wb_issue(j0 + 1, 1)

            @pl.when(g < npairs - 1)
            def _():
                idx_load(j0 + 3, 1)
            return carry

        lax.fori_loop(0, npairs, pair, 0)
        wb_wait(1)
        wb_wait(0)
        if tailc:
            tb = pl.multiple_of(base0 + nfull * _CC, 8)
            pltpu.sync_copy(send_hbm.at[pl.ds(tb, tailc)], idx_s.at[0, pl.ds(0, tailc)])
            pltpu.sync_copy(recv_hbm.at[pl.ds(tb, tailc)], idx_r.at[0, pl.ds(0, tailc)])
            cpa = pltpu.async_copy(xs_hbm.at[idx_s.at[0, pl.ds(0, tailc)]],
                                   buf_a.at[0, pl.ds(0, tailc)], sem_g0)
            cpb = pltpu.async_copy(xr_hbm.at[idx_r.at[0, pl.ds(0, tailc)]],
                                   buf_b.at[0, pl.ds(0, tailc)], sem_g1)
            cpa.wait()
            cpb.wait()
            pltpu.sync_copy(buf_a.at[0, pl.ds(0, tailc)], a_hbm.at[pl.ds(tb, tailc)])
            pltpu.sync_copy(buf_b.at[0, pl.ds(0, tailc)], b_hbm.at[pl.ds(tb, tailc)])

    return k(xw_s, xw_r, send, recv)


def _sc_counts(recv, zeros_nd, ones_cd):
    e = recv.shape[0]
    n, d = zeros_nd.shape
    ew = e // _NW
    nfull = ew // _CC
    tailc = ew - nfull * _CC
    npairs = nfull // 2
    rt = (n // _NS) & ~7
    tail = n - rt * _NS
    assert nfull % 2 == 0 and tailc % 8 == 0 and rt % 8 == 0 and tail % 8 == 0
    mesh = plsc.VectorSubcoreMesh(core_axis_name="c", subcore_axis_name="s")

    @functools.partial(
        pl.kernel,
        out_type=jax.ShapeDtypeStruct((_NC * n, d), jnp.float32),
        mesh=mesh,
        scratch_types=[
            pltpu.VMEM((2, _CC), jnp.int32),
            pltpu.VMEM((_CC, d), jnp.float32),
            pltpu.VMEM_SHARED((n + 16, d), jnp.float32),
            pltpu.SemaphoreType.DMA,
            pltpu.SemaphoreType.DMA,
        ],
    )
    def k(recv_hbm, zs_hbm, ones_hbm, pcnt_hbm, idx2, ones_v, cacc, sem0, sem1):
        c = lax.axis_index("c")
        s = lax.axis_index("s")
        wid = s * _NC + c
        base0 = pl.multiple_of(wid * ew, 8)
        row0 = pl.multiple_of(s * rt, 8)
        out0 = pl.multiple_of(c * n, 8)
        sems = (sem0, sem1)

        def each_acc_slab(fn):
            fn(row0, rt)
            if tail:
                @pl.when(s == _NS - 1)
                def _():
                    fn(rt * _NS, tail)

        each_acc_slab(lambda r0, sz: pltpu.sync_copy(
            zs_hbm.at[pl.ds(r0, sz)], cacc.at[pl.ds(r0, sz)]))
        pltpu.sync_copy(ones_hbm, ones_v)
        plsc.subcore_barrier()

        def idx_load(j, sl):
            pltpu.async_copy(recv_hbm.at[pl.ds(pl.multiple_of(base0 + j * _CC, 8), _CC)],
                             idx2.at[sl], sems[sl])

        def idx_wait(sl):
            pltpu.make_async_copy(recv_hbm.at[pl.ds(base0, _CC)], idx2.at[sl], sems[sl]).wait()

        def scat(sl):
            pltpu.sync_copy(ones_v, cacc.at[idx2.at[sl]], add=True)

        idx_load(0, 0)

        def pair(g, carry):
            j0 = 2 * g
            idx_load(j0 + 1, 1)
            idx_wait(0)
            scat(0)

            @pl.when(g < npairs - 1)
            def _():
                idx_load(j0 + 2, 0)
            idx_wait(1)
            scat(1)
            return carry

        lax.fori_loop(0, npairs, pair, 0)
        if tailc:
            tb = pl.multiple_of(base0 + nfull * _CC, 8)
            pltpu.sync_copy(recv_hbm.at[pl.ds(tb, tailc)], idx2.at[0, pl.ds(0, tailc)])
            for t in range(tailc // 16, _CC // 16):
                idx2[0, pl.ds(t * 16, 16)] = jnp.full((16,), n, jnp.int32)
            scat(0)
        plsc.subcore_barrier()
        each_acc_slab(lambda r0, sz: pltpu.sync_copy(
            cacc.at[pl.ds(r0, sz)], pcnt_hbm.at[pl.ds(out0 + r0, sz)]))

    return k(recv, zeros_nd, ones_cd).reshape(_NC, n, d)


def _edge_mlp(a, b, ea, w1e_t, b1, w2_t, b2):
    e, d = a.shape
    be = 1280
    assert e % be == 0

    def body(a_ref, b_ref, ea_ref, w1e_ref, b1_ref, w2_ref, b2_ref, h_ref):
        ea16 = ea_ref[...].astype(jnp.bfloat16)
        t = (a_ref[...] + b_ref[...] + b1_ref[...]
             + jnp.dot(ea16, w1e_ref[...], preferred_element_type=jnp.float32))
        t = t * jax.nn.sigmoid(t)
        t2 = (jnp.dot(t.astype(jnp.bfloat16), w2_ref[...],
                      preferred_element_type=jnp.float32) + b2_ref[...])
        h_ref[...] = t2 * jax.nn.sigmoid(t2)

    row = lambda i: (i, 0)
    rep = lambda i: (0, 0)
    return pl.pallas_call(
        body,
        grid=(e // be,),
        in_specs=[
            pl.BlockSpec((be, d), row),
            pl.BlockSpec((be, d), row),
            pl.BlockSpec((be, d), row),
            pl.BlockSpec((d, d), rep),
            pl.BlockSpec((1, d), rep),
            pl.BlockSpec((d, d), rep),
            pl.BlockSpec((1, d), rep),
        ],
        out_specs=pl.BlockSpec((be, d), row),
        out_shape=jax.ShapeDtypeStruct((e, d), jnp.float32),
        compiler_params=pltpu.CompilerParams(
            dimension_semantics=("arbitrary",)),
    )(a, b, ea, w1e_t.astype(jnp.bfloat16), b1.reshape(1, d),
      w2_t.astype(jnp.bfloat16), b2.reshape(1, d))


def _sc_scatter(h, recv, zsum):
    e, d = h.shape
    n = zsum.shape[0]
    ew = e // _NW
    rt = (n // _NS) & ~7
    tail = n - rt * _NS
    assert e % _NW == 0 and ew % _C == 0 and rt % 8 == 0 and tail % 8 == 0
    mesh = plsc.VectorSubcoreMesh(core_axis_name="c", subcore_axis_name="s")

    nfull = ew // _CC
    tailc = ew - nfull * _CC
    npairs = nfull // 2
    assert nfull % 2 == 0 and tailc % 8 == 0

    @functools.partial(
        pl.kernel,
        out_type=jax.ShapeDtypeStruct((_NC * n, d), jnp.float32),
        mesh=mesh,
        scratch_types=[
            pltpu.VMEM((2, _CC), jnp.int32),
            pltpu.VMEM((2, _CC, d), jnp.float32),
            pltpu.VMEM_SHARED((n + 16, d), jnp.float32),
            pltpu.SemaphoreType.DMA,
            pltpu.SemaphoreType.DMA,
        ],
    )
    def k(h_hbm, recv_hbm, zs_hbm, psum_hbm, idx2, buf, acc, sem0, sem1):
        c = lax.axis_index("c")
        s = lax.axis_index("s")
        wid = s * _NC + c
        base0 = pl.multiple_of(wid * ew, 8)
        row0 = pl.multiple_of(s * rt, 8)
        out0 = pl.multiple_of(c * n, 8)
        sems = (sem0, sem1)

        def each_acc_slab(fn):
            fn(row0, rt)
            if tail:
                @pl.when(s == _NS - 1)
                def _():
                    fn(rt * _NS, tail)

        each_acc_slab(lambda r0, sz: pltpu.sync_copy(
            zs_hbm.at[pl.ds(r0, sz)], acc.at[pl.ds(r0, sz)]))
        plsc.subcore_barrier()

        def load(j, sl):
            base = pl.multiple_of(base0 + j * _CC, 8)
            pltpu.async_copy(recv_hbm.at[pl.ds(base, _CC)], idx2.at[sl], sems[sl])
            pltpu.async_copy(h_hbm.at[pl.ds(base, _CC)], buf.at[sl], sems[sl])

        def load_wait(sl):
            pltpu.make_async_copy(recv_hbm.at[pl.ds(base0, _CC)], idx2.at[sl], sems[sl]).wait()
            pltpu.make_async_copy(h_hbm.at[pl.ds(base0, _CC)], buf.at[sl], sems[sl]).wait()

        def scat(sl):
            pltpu.sync_copy(buf.at[sl], acc.at[idx2.at[sl]], add=True)

        load(0, 0)

        def pair(g, carry):
            j0 = 2 * g
            load(j0 + 1, 1)
            load_wait(0)
            scat(0)

            @pl.when(g < npairs - 1)
            def _():
                load(j0 + 2, 0)
            load_wait(1)
            scat(1)
            return carry

        lax.fori_loop(0, npairs, pair, 0)
        if tailc:
            tb = pl.multiple_of(base0 + nfull * _CC, 8)
            pltpu.sync_copy(recv_hbm.at[pl.ds(tb, tailc)], idx2.at[0, pl.ds(0, tailc)])
            pltpu.sync_copy(h_hbm.at[pl.ds(tb, tailc)], buf.at[0, pl.ds(0, tailc)])
            for t in range(tailc // 16, _CC // 16):
                idx2[0, pl.ds(t * 16, 16)] = jnp.full((16,), n, jnp.int32)
            scat(0)
        plsc.subcore_barrier()
        each_acc_slab(lambda r0, sz: pltpu.sync_copy(
            acc.at[pl.ds(r0, sz)], psum_hbm.at[pl.ds(out0 + r0, sz)]))

    return k(h, recv, zsum).reshape(_NC, n, d)


def _update(x, psum, pcnt, u1_t, ub1, u2_t, ub2):
    n, d = x.shape

    def body(x_ref, ps_ref, pc_ref, u1_ref, ub1_ref, u2_ref, ub2_ref, out_ref):
        cnt = pc_ref[0, :, 0:1] + pc_ref[1, :, 0:1]
        mean = (ps_ref[0] + ps_ref[1]) / jnp.maximum(cnt, 1.0)
        x_new = x_ref[...] + mean
        t = jnp.dot(x_new, u1_ref[...], preferred_element_type=jnp.float32) + ub1_ref[...]
        t = t * jax.nn.sigmoid(t)
        u = jnp.dot(t, u2_ref[...], preferred_element_type=jnp.float32) + ub2_ref[...]
        out_ref[...] = x_new + u

    return pl.pallas_call(
        body,
        out_shape=jax.ShapeDtypeStruct((n, d), jnp.float32),
    )(x, psum, pcnt, u1_t, ub1.reshape(1, -1), u2_t, ub2.reshape(1, -1))


def kernel(x, edge_attr, edges, W1, b1, W2, b2, U1, ub1, U2, ub2):
    n, d = x.shape
    send = edges[0]
    recv = edges[1]
    w1s_t = W1[:, :d].T
    w1r_t = W1[:, d:2 * d].T
    w1e_t = W1[:, 2 * d:].T
    w2_t = W2.T
    u1_t = U1.T
    u2_t = U2.T

    xw_s, xw_r = _premul(x, w1s_t, w1r_t)
    zsum = jnp.zeros((n, d), jnp.float32)
    ones_cd = jnp.ones((_CC, d), jnp.float32)
    a, b = _sc_gather(xw_s, xw_r, send, recv)
    h = _edge_mlp(a, b, edge_attr, w1e_t, b1, w2_t, b2)
    pcnt = _sc_counts(recv, zsum, ones_cd)
    psum = _sc_scatter(h, recv, zsum)
    x_out = _update(x, psum, pcnt, u1_t, ub1, u2_t, ub2)
    return (x_out, h)


# speedup vs baseline: 1.1260x; 1.1177x over previous
import functools

import jax
import jax.numpy as jnp
from jax import lax
from jax.experimental import pallas as pl
from jax.experimental.pallas import tpu as pltpu
from jax.experimental.pallas import tpu_sc as plsc

_NC = 2
_NS = 16
_NW = _NC * _NS
_C = 80


def _premul(x, ws_t, wr_t):
    n, d = x.shape

    def body(x_ref, ws_ref, wr_ref, a_ref, b_ref):
        xv = x_ref[...]
        a_ref[...] = jnp.dot(xv, ws_ref[...], preferred_element_type=jnp.float32)
        b_ref[...] = jnp.dot(xv, wr_ref[...], preferred_element_type=jnp.float32)

    return pl.pallas_call(
        body,
        out_shape=[jax.ShapeDtypeStruct((n, d), jnp.float32),
                   jax.ShapeDtypeStruct((n, d), jnp.float32)],
    )(x, ws_t, wr_t)


_CC = 128


def _sc_gather(xw_s, xw_r, send, recv):
    e = send.shape[0]
    d = xw_s.shape[1]
    ew = e // _NW
    nfull = ew // _CC
    tailc = ew - nfull * _CC
    npairs = nfull // 2
    assert e % _NW == 0 and nfull % 2 == 0 and tailc % 8 == 0
    mesh = plsc.VectorSubcoreMesh(core_axis_name="c", subcore_axis_name="s")

    @functools.partial(
        pl.kernel,
        out_type=jax.ShapeDtypeStruct((e, d), jnp.float32),
        mesh=mesh,
        scratch_types=[
            pltpu.VMEM((2, _CC), jnp.int32),
            pltpu.VMEM((2, _CC), jnp.int32),
            pltpu.VMEM((2, _CC, d), jnp.float32),
            pltpu.VMEM((2, _CC, d), jnp.float32),
            pltpu.SemaphoreType.DMA,
            pltpu.SemaphoreType.DMA,
            pltpu.SemaphoreType.DMA,
            pltpu.SemaphoreType.DMA,
            pltpu.SemaphoreType.DMA,
        ],
    )
    def k(xs_hbm, xr_hbm, send_hbm, recv_hbm, g_hbm,
          idx_s, idx_r, buf_a, buf_b, sem_i, sem_g0, sem_g1, sem_w0, sem_w1):
        wid = lax.axis_index("s") * _NC + lax.axis_index("c")
        base0 = pl.multiple_of(wid * ew, 8)
        semw = (sem_w0, sem_w1)
        semg = (sem_g0, sem_g1)

        def jbase(j):
            return pl.multiple_of(base0 + j * _CC, 8)

        def idx_load(j, sl):
            pltpu.async_copy(send_hbm.at[pl.ds(jbase(j), _CC)], idx_s.at[sl], sem_i)
            pltpu.async_copy(recv_hbm.at[pl.ds(jbase(j), _CC)], idx_r.at[sl], sem_i)

        def idx_wait(sl):
            pltpu.make_async_copy(send_hbm.at[pl.ds(base0, _CC)], idx_s.at[sl], sem_i).wait()
            pltpu.make_async_copy(recv_hbm.at[pl.ds(base0, _CC)], idx_r.at[sl], sem_i).wait()

        def gather_issue(sl):
            pltpu.async_copy(xs_hbm.at[idx_s.at[sl]], buf_a.at[sl], semg[sl])
            pltpu.async_copy(xr_hbm.at[idx_r.at[sl]], buf_b.at[sl], semg[sl])

        def gather_wait(sl):
            pltpu.make_async_copy(xs_hbm.at[idx_s.at[sl]], buf_a.at[sl], semg[sl]).wait()
            pltpu.make_async_copy(xr_hbm.at[idx_r.at[sl]], buf_b.at[sl], semg[sl]).wait()

        def wb_issue(j, sl):
            pltpu.async_copy(buf_a.at[sl], g_hbm.at[pl.ds(jbase(j), _CC)], semw[sl])

        def wb_wait(sl):
            pltpu.make_async_copy(buf_a.at[sl], g_hbm.at[pl.ds(base0, _CC)], semw[sl]).wait()

        def add_rows(sl, nrows):
            def row(r, carry):
                for cc in range(d // 16):
                    buf_a[sl, r, pl.ds(cc * 16, 16)] = (
                        buf_a[sl, r, pl.ds(cc * 16, 16)]
                        + buf_b[sl, r, pl.ds(cc * 16, 16)])
                return carry
            lax.fori_loop(0, nrows, row, 0)

        idx_load(0, 0)
        idx_wait(0)
        gather_issue(0)
        idx_load(1, 1)

        def pair(g, carry):
            j0 = 2 * g
            idx_wait(1)

            @pl.when(g > 0)
            def _():
                wb_wait(1)
            gather_issue(1)
            gather_wait(0)
            add_rows(0, _CC)
            wb_issue(j0, 0)

            @pl.when(g < npairs - 1)
            def _():
                idx_load(j0 + 2, 0)
                idx_wait(0)
                wb_wait(0)
                gather_issue(0)
            gather_wait(1)
            add_rows(1, _CC)
            wb_issue(j0 + 1, 1)

            @pl.when(g < npairs - 1)
            def _():
                idx_load(j0 + 3, 1)
            return carry

        lax.fori_loop(0, npairs, pair, 0)
        wb_wait(1)
        wb_wait(0)
        if tailc:
            tb = pl.multiple_of(base0 + nfull * _CC, 8)
            pltpu.sync_copy(send_hbm.at[pl.ds(tb, tailc)], idx_s.at[0, pl.ds(0, tailc)])
            pltpu.sync_copy(recv_hbm.at[pl.ds(tb, tailc)], idx_r.at[0, pl.ds(0, tailc)])
            cpa = pltpu.async_copy(xs_hbm.at[idx_s.at[0, pl.ds(0, tailc)]],
                                   buf_a.at[0, pl.ds(0, tailc)], sem_g0)
            cpb = pltpu.async_copy(xr_hbm.at[idx_r.at[0, pl.ds(0, tailc)]],
                                   buf_b.at[0, pl.ds(0, tailc)], sem_g1)
            cpa.wait()
            cpb.wait()
            add_rows(0, tailc)
            pltpu.sync_copy(buf_a.at[0, pl.ds(0, tailc)], g_hbm.at[pl.ds(tb, tailc)])

    return k(xw_s, xw_r, send, recv)


def _sc_counts(recv, zeros_nd, ones_cd):
    e = recv.shape[0]
    n, d = zeros_nd.shape
    ew = e // _NW
    nfull = ew // _CC
    tailc = ew - nfull * _CC
    npairs = nfull // 2
    rt = (n // _NS) & ~7
    tail = n - rt * _NS
    assert nfull % 2 == 0 and tailc % 8 == 0 and rt % 8 == 0 and tail % 8 == 0
    mesh = plsc.VectorSubcoreMesh(core_axis_name="c", subcore_axis_name="s")

    @functools.partial(
        pl.kernel,
        out_type=jax.ShapeDtypeStruct((_NC * n, d), jnp.float32),
        mesh=mesh,
        scratch_types=[
            pltpu.VMEM((2, _CC), jnp.int32),
            pltpu.VMEM((_CC, d), jnp.float32),
            pltpu.VMEM_SHARED((n + 16, d), jnp.float32),
            pltpu.SemaphoreType.DMA,
            pltpu.SemaphoreType.DMA,
        ],
    )
    def k(recv_hbm, zs_hbm, ones_hbm, pcnt_hbm, idx2, ones_v, cacc, sem0, sem1):
        c = lax.axis_index("c")
        s = lax.axis_index("s")
        wid = s * _NC + c
        base0 = pl.multiple_of(wid * ew, 8)
        row0 = pl.multiple_of(s * rt, 8)
        out0 = pl.multiple_of(c * n, 8)
        sems = (sem0, sem1)

        def each_acc_slab(fn):
            fn(row0, rt)
            if tail:
                @pl.when(s == _NS - 1)
                def _():
                    fn(rt * _NS, tail)

        each_acc_slab(lambda r0, sz: pltpu.sync_copy(
            zs_hbm.at[pl.ds(r0, sz)], cacc.at[pl.ds(r0, sz)]))
        pltpu.sync_copy(ones_hbm, ones_v)
        plsc.subcore_barrier()

        def idx_load(j, sl):
            pltpu.async_copy(recv_hbm.at[pl.ds(pl.multiple_of(base0 + j * _CC, 8), _CC)],
                             idx2.at[sl], sems[sl])

        def idx_wait(sl):
            pltpu.make_async_copy(recv_hbm.at[pl.ds(base0, _CC)], idx2.at[sl], sems[sl]).wait()

        def scat(sl):
            pltpu.sync_copy(ones_v, cacc.at[idx2.at[sl]], add=True)

        idx_load(0, 0)

        def pair(g, carry):
            j0 = 2 * g
            idx_load(j0 + 1, 1)
            idx_wait(0)
            scat(0)

            @pl.when(g < npairs - 1)
            def _():
                idx_load(j0 + 2, 0)
            idx_wait(1)
            scat(1)
            return carry

        lax.fori_loop(0, npairs, pair, 0)
        if tailc:
            tb = pl.multiple_of(base0 + nfull * _CC, 8)
            pltpu.sync_copy(recv_hbm.at[pl.ds(tb, tailc)], idx2.at[0, pl.ds(0, tailc)])
            for t in range(tailc // 16, _CC // 16):
                idx2[0, pl.ds(t * 16, 16)] = jnp.full((16,), n, jnp.int32)
            scat(0)
        plsc.subcore_barrier()
        each_acc_slab(lambda r0, sz: pltpu.sync_copy(
            cacc.at[pl.ds(r0, sz)], pcnt_hbm.at[pl.ds(out0 + r0, sz)]))

    return k(recv, zeros_nd, ones_cd).reshape(_NC, n, d)


def _edge_mlp(g, ea, w1e_t, b1, w2_t, b2):
    e, d = g.shape
    be = 1280
    assert e % be == 0

    def body(g_ref, ea_ref, w1e_ref, b1_ref, w2_ref, b2_ref, h_ref):
        ea16 = ea_ref[...].astype(jnp.bfloat16)
        t = (g_ref[...] + b1_ref[...]
             + jnp.dot(ea16, w1e_ref[...], preferred_element_type=jnp.float32))
        t = t * jax.nn.sigmoid(t)
        t2 = (jnp.dot(t.astype(jnp.bfloat16), w2_ref[...],
                      preferred_element_type=jnp.float32) + b2_ref[...])
        h_ref[...] = t2 * jax.nn.sigmoid(t2)

    row = lambda i: (i, 0)
    rep = lambda i: (0, 0)
    return pl.pallas_call(
        body,
        grid=(e // be,),
        in_specs=[
            pl.BlockSpec((be, d), row),
            pl.BlockSpec((be, d), row),
            pl.BlockSpec((d, d), rep),
            pl.BlockSpec((1, d), rep),
            pl.BlockSpec((d, d), rep),
            pl.BlockSpec((1, d), rep),
        ],
        out_specs=pl.BlockSpec((be, d), row),
        out_shape=jax.ShapeDtypeStruct((e, d), jnp.float32),
        compiler_params=pltpu.CompilerParams(
            dimension_semantics=("arbitrary",)),
    )(g, ea, w1e_t.astype(jnp.bfloat16), b1.reshape(1, d),
      w2_t.astype(jnp.bfloat16), b2.reshape(1, d))


def _sc_scatter(h, recv, zsum):
    e, d = h.shape
    n = zsum.shape[0]
    ew = e // _NW
    rt = (n // _NS) & ~7
    tail = n - rt * _NS
    assert e % _NW == 0 and ew % _C == 0 and rt % 8 == 0 and tail % 8 == 0
    mesh = plsc.VectorSubcoreMesh(core_axis_name="c", subcore_axis_name="s")

    nfull = ew // _CC
    tailc = ew - nfull * _CC
    npairs = nfull // 2
    assert nfull % 2 == 0 and tailc % 8 == 0

    @functools.partial(
        pl.kernel,
        out_type=jax.ShapeDtypeStruct((_NC * n, d), jnp.float32),
        mesh=mesh,
        scratch_types=[
            pltpu.VMEM((2, _CC), jnp.int32),
            pltpu.VMEM((2, _CC, d), jnp.float32),
            pltpu.VMEM_SHARED((n + 16, d), jnp.float32),
            pltpu.SemaphoreType.DMA,
            pltpu.SemaphoreType.DMA,
        ],
    )
    def k(h_hbm, recv_hbm, zs_hbm, psum_hbm, idx2, buf, acc, sem0, sem1):
        c = lax.axis_index("c")
        s = lax.axis_index("s")
        wid = s * _NC + c
        base0 = pl.multiple_of(wid * ew, 8)
        row0 = pl.multiple_of(s * rt, 8)
        out0 = pl.multiple_of(c * n, 8)
        sems = (sem0, sem1)

        def each_acc_slab(fn):
            fn(row0, rt)
            if tail:
                @pl.when(s == _NS - 1)
                def _():
                    fn(rt * _NS, tail)

        each_acc_slab(lambda r0, sz: pltpu.sync_copy(
            zs_hbm.at[pl.ds(r0, sz)], acc.at[pl.ds(r0, sz)]))
        plsc.subcore_barrier()

        def load(j, sl):
            base = pl.multiple_of(base0 + j * _CC, 8)
            pltpu.async_copy(recv_hbm.at[pl.ds(base, _CC)], idx2.at[sl], sems[sl])
            pltpu.async_copy(h_hbm.at[pl.ds(base, _CC)], buf.at[sl], sems[sl])

        def load_wait(sl):
            pltpu.make_async_copy(recv_hbm.at[pl.ds(base0, _CC)], idx2.at[sl], sems[sl]).wait()
            pltpu.make_async_copy(h_hbm.at[pl.ds(base0, _CC)], buf.at[sl], sems[sl]).wait()

        def scat(sl):
            pltpu.sync_copy(buf.at[sl], acc.at[idx2.at[sl]], add=True)

        load(0, 0)

        def pair(g, carry):
            j0 = 2 * g
            load(j0 + 1, 1)
            load_wait(0)
            scat(0)

            @pl.when(g < npairs - 1)
            def _():
                load(j0 + 2, 0)
            load_wait(1)
            scat(1)
            return carry

        lax.fori_loop(0, npairs, pair, 0)
        if tailc:
            tb = pl.multiple_of(base0 + nfull * _CC, 8)
            pltpu.sync_copy(recv_hbm.at[pl.ds(tb, tailc)], idx2.at[0, pl.ds(0, tailc)])
            pltpu.sync_copy(h_hbm.at[pl.ds(tb, tailc)], buf.at[0, pl.ds(0, tailc)])
            for t in range(tailc // 16, _CC // 16):
                idx2[0, pl.ds(t * 16, 16)] = jnp.full((16,), n, jnp.int32)
            scat(0)
        plsc.subcore_barrier()
        each_acc_slab(lambda r0, sz: pltpu.sync_copy(
            acc.at[pl.ds(r0, sz)], psum_hbm.at[pl.ds(out0 + r0, sz)]))

    return k(h, recv, zsum).reshape(_NC, n, d)


def _update(x, psum, pcnt, u1_t, ub1, u2_t, ub2):
    n, d = x.shape

    def body(x_ref, ps_ref, pc_ref, u1_ref, ub1_ref, u2_ref, ub2_ref, out_ref):
        cnt = pc_ref[0, :, 0:1] + pc_ref[1, :, 0:1]
        mean = (ps_ref[0] + ps_ref[1]) / jnp.maximum(cnt, 1.0)
        x_new = x_ref[...] + mean
        t = jnp.dot(x_new, u1_ref[...], preferred_element_type=jnp.float32) + ub1_ref[...]
        t = t * jax.nn.sigmoid(t)
        u = jnp.dot(t, u2_ref[...], preferred_element_type=jnp.float32) + ub2_ref[...]
        out_ref[...] = x_new + u

    return pl.pallas_call(
        body,
        out_shape=jax.ShapeDtypeStruct((n, d), jnp.float32),
    )(x, psum, pcnt, u1_t, ub1.reshape(1, -1), u2_t, ub2.reshape(1, -1))


def kernel(x, edge_attr, edges, W1, b1, W2, b2, U1, ub1, U2, ub2):
    n, d = x.shape
    send = edges[0]
    recv = edges[1]
    w1s_t = W1[:, :d].T
    w1r_t = W1[:, d:2 * d].T
    w1e_t = W1[:, 2 * d:].T
    w2_t = W2.T
    u1_t = U1.T
    u2_t = U2.T

    xw_s, xw_r = _premul(x, w1s_t, w1r_t)
    zsum = jnp.zeros((n, d), jnp.float32)
    ones_cd = jnp.ones((_CC, d), jnp.float32)
    g = _sc_gather(xw_s, xw_r, send, recv)
    h = _edge_mlp(g, edge_attr, w1e_t, b1, w2_t, b2)
    pcnt = _sc_counts(recv, zsum, ones_cd)
    psum = _sc_scatter(h, recv, zsum)
    x_out = _update(x, psum, pcnt, u1_t, ub1, u2_t, ub2)
    return (x_out, h)
